# baseline (speedup 1.0000x reference)
import jax
import jax.numpy as jnp
from jax import lax
from jax.experimental import pallas as pl
from jax.experimental.pallas import tpu as pltpu
from jax.experimental.pallas import tpu_sc as plsc

B = 16384
S = 200
D = 64
H = S // 2
NC = 2
NS = 16
NW = NC * NS
RPW = B // NW
CR = 32
NCHUNK = RPW // CR
SPC = CR * S
NG = SPC // 16
HPC = SPC // 2

_DNUMS = lax.GatherDimensionNumbers(
    offset_dims=(), collapsed_slice_dims=(0,), start_index_map=(0,))


def _dg(v, idx):
    return lax.gather(v, idx[:, None], _DNUMS, slice_sizes=(1,),
                      mode=lax.GatherScatterMode.PROMISE_IN_BOUNDS)


def _par_body(x_hbm, pe_hbm, po_hbm, xb_a, xb_b, pe_a, po_a, pe_b, po_b,
              sx_a, sx_b, sp_a, sp_b):
    cid = lax.axis_index("c")
    sid = lax.axis_index("s")
    wid = sid * NC + cid
    tok_base = wid * RPW * S
    lane = lax.iota(jnp.int32, 16)
    zero16 = lane * 0
    even = (lane & 7) * 2
    odd = even + 1

    def x_slice(c):
        off = pl.multiple_of(tok_base + c * SPC, 32)
        return x_hbm.at[pl.ds(off, SPC)]

    def p_slice(hbm, c):
        off = pl.multiple_of((tok_base + c * SPC) // 2, 32)
        return hbm.at[pl.ds(off, HPC)]

    def compute_chunk(xb, pe, po):
        def scan_group(g, carry):
            v = xb[pl.ds(g * 16, 16)]
            scan = plsc.cumsum(v)
            t = scan + carry
            ex = t - v
            gm = g % 25
            lam = jnp.where(gm == 0, 0, jnp.where(gm == 12, 8, 16))
            sub = _dg(ex, zero16 + jnp.minimum(lam, 15))
            tot = jnp.where(lane < lam, t, t - sub)
            return tot & 1, _dg(tot, zero16 + 15)

        def gpair(g2, carry):
            pa, carry = scan_group(2 * g2, carry)
            pb, carry = scan_group(2 * g2 + 1, carry)
            ev = jnp.where(lane < 8, _dg(pa, even), _dg(pb, even))
            od = jnp.where(lane < 8, _dg(pa, odd), _dg(pb, odd))
            pe[pl.ds(g2 * 16, 16)] = ev.astype(jnp.float32)
            po[pl.ds(g2 * 16, 16)] = od.astype(jnp.float32)
            return carry

        lax.fori_loop(0, NG // 2, gpair, jnp.zeros((16,), jnp.int32))

    pltpu.async_copy(x_slice(0), xb_a, sx_a)
    pltpu.async_copy(x_slice(1), xb_b, sx_b)

    bufs = ((xb_a, pe_a, po_a, sx_a, sp_a), (xb_b, pe_b, po_b, sx_b, sp_b))

    def step(i, carry_unused):
        for j, (xb, pe, po, sx, sp) in enumerate(bufs):
            c = 2 * i + j
            pltpu.make_async_copy(x_slice(0), xb, sx).wait()

            @pl.when(i > 0)
            def _():
                pltpu.make_async_copy(pe, p_slice(pe_hbm, 0), sp).wait()
                pltpu.make_async_copy(po, p_slice(po_hbm, 0), sp).wait()

            compute_chunk(xb, pe, po)
            pltpu.async_copy(pe, p_slice(pe_hbm, c), sp)
            pltpu.async_copy(po, p_slice(po_hbm, c), sp)

            @pl.when(c + 2 < NCHUNK)
            def _():
                pltpu.async_copy(x_slice(c + 2), xb, sx)
        return carry_unused

    lax.fori_loop(0, NCHUNK // 2, step, 0)
    for pe, po, sp in ((pe_a, po_a, sp_a), (pe_b, po_b, sp_b)):
        pltpu.make_async_copy(pe, p_slice(pe_hbm, 0), sp).wait()
        pltpu.make_async_copy(po, p_slice(po_hbm, 0), sp).wait()


_sc_par = pl.kernel(
    _par_body,
    out_type=(jax.ShapeDtypeStruct((B * H,), jnp.float32),
              jax.ShapeDtypeStruct((B * H,), jnp.float32)),
    mesh=plsc.VectorSubcoreMesh(core_axis_name="c", subcore_axis_name="s"),
    compiler_params=pltpu.CompilerParams(needs_layout_passes=False),
    scratch_types=[
        pltpu.VMEM((SPC,), jnp.int32),
        pltpu.VMEM((SPC,), jnp.int32),
        pltpu.VMEM((HPC,), jnp.float32),
        pltpu.VMEM((HPC,), jnp.float32),
        pltpu.VMEM((HPC,), jnp.float32),
        pltpu.VMEM((HPC,), jnp.float32),
        pltpu.SemaphoreType.DMA,
        pltpu.SemaphoreType.DMA,
        pltpu.SemaphoreType.DMA,
        pltpu.SemaphoreType.DMA,
    ],
)

BB = 128
NTB = B // BB


def _tc_body(pe_ref, po_ref, tab_ref, o_hbm, buf_a, buf_b, sem_a, sem_b):
    i = pl.program_id(0)
    t0 = tab_ref[0, :]
    diff = tab_ref[1, :] - t0

    def run(buf, sem, other_buf, other_sem):
        @pl.when(i >= 2)
        def _():
            pltpu.make_async_copy(buf.at[pl.ds(0, BB // 8)], o_hbm.at[pl.ds(0, BB // 8)], sem).wait()

        he = pe_ref[...][:, :, None] * diff[None, None, :] + t0[None, None, :]
        ho = po_ref[...][:, :, None] * diff[None, None, :] + t0[None, None, :]
        buf[...] = jnp.concatenate([he, ho], axis=-1)
        pltpu.async_copy(buf.at[pl.ds(0, BB // 8)],
                         o_hbm.at[pl.ds(i * BB, BB // 8)], sem)

        @pl.when(i == NTB - 1)
        def _():
            pltpu.make_async_copy(other_buf.at[pl.ds(0, BB // 8)], o_hbm.at[pl.ds(0, BB // 8)], other_sem).wait()
            pltpu.make_async_copy(buf.at[pl.ds(0, BB // 8)], o_hbm.at[pl.ds(0, BB // 8)], sem).wait()

    @pl.when(i % 2 == 0)
    def _():
        run(buf_a, sem_a, buf_b, sem_b)

    @pl.when(i % 2 == 1)
    def _():
        run(buf_b, sem_b, buf_a, sem_a)


_tc_expand = pl.pallas_call(
    _tc_body,
    grid=(NTB,),
    in_specs=[
        pl.BlockSpec((BB, H), lambda i: (i, 0)),
        pl.BlockSpec((BB, H), lambda i: (i, 0)),
        pl.BlockSpec((2, D), lambda i: (0, 0)),
    ],
    out_specs=pl.BlockSpec(memory_space=pltpu.MemorySpace.HBM),
    out_shape=jax.ShapeDtypeStruct((B, H, 2 * D), jnp.float32),
    scratch_shapes=[
        pltpu.VMEM((BB, H, 2 * D), jnp.float32),
        pltpu.VMEM((BB, H, 2 * D), jnp.float32),
        pltpu.SemaphoreType.DMA,
        pltpu.SemaphoreType.DMA,
    ],
)


def kernel(x, table):
    pe, po = _sc_par(x.reshape(-1))
    out = _tc_expand(pe.reshape(B, H), po.reshape(B, H), table)
    return out.reshape(B, S, D)


# speedup vs baseline: 1.3518x; 1.3518x over previous
import jax
import jax.numpy as jnp
from jax import lax
from jax.experimental import pallas as pl
from jax.experimental.pallas import tpu as pltpu
from jax.experimental.pallas import tpu_sc as plsc

B = 16384
S = 200
D = 64
H = S // 2
NC = 2
NS = 16
NW = NC * NS
RPW = B // NW
CR = 32
NCHUNK = RPW // CR
SPC = CR * S
NG = SPC // 16
HPC = SPC // 2

_DNUMS = lax.GatherDimensionNumbers(
    offset_dims=(), collapsed_slice_dims=(0,), start_index_map=(0,))


def _dg(v, idx):
    return lax.gather(v, idx[:, None], _DNUMS, slice_sizes=(1,),
                      mode=lax.GatherScatterMode.PROMISE_IN_BOUNDS)


def _par_body(x_hbm, q_hbm, xb_a, xb_b, qv_a, qv_b,
              sx_a, sx_b, sp_a, sp_b):
    cid = lax.axis_index("c")
    sid = lax.axis_index("s")
    wid = sid * NC + cid
    tok_base = wid * RPW * S
    lane = lax.iota(jnp.int32, 16)
    zero16 = lane * 0
    even = (lane & 7) * 2
    odd = even + 1

    def x_slice(c):
        off = pl.multiple_of(tok_base + c * SPC, 32)
        return x_hbm.at[pl.ds(off, SPC)]

    def p_slice(hbm, c):
        off = pl.multiple_of((tok_base + c * SPC) // 2, 32)
        return hbm.at[pl.ds(off, HPC)]

    def compute_chunk(xb, qv):
        def scan_group(g, carry):
            v = xb[pl.ds(g * 16, 16)]
            scan = plsc.cumsum(v)
            t = scan + carry
            ex = t - v
            gm = g % 25
            lam = jnp.where(gm == 0, 0, jnp.where(gm == 12, 8, 16))
            sub = _dg(ex, zero16 + jnp.minimum(lam, 15))
            tot = jnp.where(lane < lam, t, t - sub)
            return tot & 1, _dg(tot, zero16 + 15)

        def gpair(g2, carry):
            pa, carry = scan_group(2 * g2, carry)
            pb, carry = scan_group(2 * g2 + 1, carry)
            ev = jnp.where(lane < 8, _dg(pa, even), _dg(pb, even))
            od = jnp.where(lane < 8, _dg(pa, odd), _dg(pb, odd))
            qv[pl.ds(g2 * 16, 16)] = ev * 2 + od
            return carry

        lax.fori_loop(0, NG // 2, gpair, jnp.zeros((16,), jnp.int32))

    pltpu.async_copy(x_slice(0), xb_a, sx_a)
    pltpu.async_copy(x_slice(1), xb_b, sx_b)

    bufs = ((xb_a, qv_a, sx_a, sp_a), (xb_b, qv_b, sx_b, sp_b))

    def step(i, carry_unused):
        for j, (xb, qv, sx, sp) in enumerate(bufs):
            c = 2 * i + j
            pltpu.make_async_copy(x_slice(0), xb, sx).wait()

            @pl.when(i > 0)
            def _():
                pltpu.make_async_copy(qv, p_slice(q_hbm, 0), sp).wait()

            compute_chunk(xb, qv)
            pltpu.async_copy(qv, p_slice(q_hbm, c), sp)

            @pl.when(c + 2 < NCHUNK)
            def _():
                pltpu.async_copy(x_slice(c + 2), xb, sx)
        return carry_unused

    lax.fori_loop(0, NCHUNK // 2, step, 0)
    for qv, sp in ((qv_a, sp_a), (qv_b, sp_b)):
        pltpu.make_async_copy(qv, p_slice(q_hbm, 0), sp).wait()


_sc_par = pl.kernel(
    _par_body,
    out_type=jax.ShapeDtypeStruct((B * H,), jnp.int32),
    mesh=plsc.VectorSubcoreMesh(core_axis_name="c", subcore_axis_name="s"),
    compiler_params=pltpu.CompilerParams(needs_layout_passes=False),
    scratch_types=[
        pltpu.VMEM((SPC,), jnp.int32),
        pltpu.VMEM((SPC,), jnp.int32),
        pltpu.VMEM((HPC,), jnp.int32),
        pltpu.VMEM((HPC,), jnp.int32),
        pltpu.SemaphoreType.DMA,
        pltpu.SemaphoreType.DMA,
        pltpu.SemaphoreType.DMA,
        pltpu.SemaphoreType.DMA,
    ],
)

BB = 128
NTB = B // BB


def _tc_body(q_ref, tab_ref, o_hbm, buf_a, buf_b, sem_a, sem_b):
    i = pl.program_id(0)
    t0 = tab_ref[0, :]
    diff = tab_ref[1, :] - t0
    t02 = jnp.concatenate([t0, t0], axis=-1)
    diff2 = jnp.concatenate([diff, diff], axis=-1)

    def run(buf, sem, other_buf, other_sem):
        @pl.when(i >= 2)
        def _():
            pltpu.make_async_copy(buf, o_hbm.at[pl.ds(0, BB)], sem).wait()

        lane128 = lax.broadcasted_iota(jnp.int32, (BB, H, 2 * D), 2)
        qs = q_ref[...][:, :, None]
        pq = jnp.where(lane128 < D, qs >> 1, qs & 1).astype(jnp.float32)
        buf[...] = pq * diff2[None, None, :] + t02[None, None, :]
        pltpu.async_copy(buf, o_hbm.at[pl.ds(i * BB, BB)], sem)

        @pl.when(i == NTB - 1)
        def _():
            pltpu.make_async_copy(other_buf, o_hbm.at[pl.ds(0, BB)], other_sem).wait()
            pltpu.make_async_copy(buf, o_hbm.at[pl.ds(0, BB)], sem).wait()

    @pl.when(i % 2 == 0)
    def _():
        run(buf_a, sem_a, buf_b, sem_b)

    @pl.when(i % 2 == 1)
    def _():
        run(buf_b, sem_b, buf_a, sem_a)


_tc_expand = pl.pallas_call(
    _tc_body,
    grid=(NTB,),
    in_specs=[
        pl.BlockSpec((BB, H), lambda i: (i, 0)),
        pl.BlockSpec((2, D), lambda i: (0, 0)),
    ],
    out_specs=pl.BlockSpec(memory_space=pltpu.MemorySpace.HBM),
    out_shape=jax.ShapeDtypeStruct((B, H, 2 * D), jnp.float32),
    scratch_shapes=[
        pltpu.VMEM((BB, H, 2 * D), jnp.float32),
        pltpu.VMEM((BB, H, 2 * D), jnp.float32),
        pltpu.SemaphoreType.DMA,
        pltpu.SemaphoreType.DMA,
    ],
)


def kernel(x, table):
    q = _sc_par(x.reshape(-1))
    out = _tc_expand(q.reshape(B, H), table)
    return out.reshape(B, S, D)
